# Initial kernel scaffold; baseline (speedup 1.0000x reference)
import functools

import jax
import jax.numpy as jnp
from jax import lax
from jax.experimental import pallas as pl
from jax.experimental.pallas import tpu as pltpu
from jax.experimental.pallas import tpu_sc as plsc

VOCAB = 1000000
D = 64
SL = 200
B = 16384
NC, NS = 2, 16
NW = NC * NS
RW = B // NW
GROUP = 8
NG = RW // GROUP
HALF = 100
IDX_PER_GROUP = GROUP * SL
NDMA = IDX_PER_GROUP // HALF

_PBLK = 8000


def _proj_body(emb_ref, w1_ref, w2_ref, t_ref):
    w = lax.dot_general(w2_ref[...], w1_ref[...], (((1,), (0,)), ((), ())),
                        preferred_element_type=jnp.float32)
    t_ref[...] = lax.dot_general(emb_ref[...], w, (((1,), (1,)), ((), ())),
                                 preferred_element_type=jnp.float32)


def _project(emb, W1, W2):
    return pl.pallas_call(
        _proj_body,
        grid=(VOCAB // _PBLK,),
        in_specs=[
            pl.BlockSpec((_PBLK, D), lambda i: (i, 0)),
            pl.BlockSpec((2 * D, D), lambda i: (0, 0)),
            pl.BlockSpec((2, 2 * D), lambda i: (0, 0)),
        ],
        out_specs=pl.BlockSpec((_PBLK, 2), lambda i: (i, 0)),
        out_shape=jax.ShapeDtypeStruct((VOCAB, 2), jnp.float32),
    )(emb, W1, W2)


def _sc_pool_softmax(t, x_flat, bvec):
    mesh = plsc.VectorSubcoreMesh(core_axis_name="c", subcore_axis_name="s")

    @functools.partial(
        pl.kernel,
        out_type=jax.ShapeDtypeStruct((B * 2,), jnp.float32),
        mesh=mesh,
        scratch_types=[
            pltpu.VMEM((IDX_PER_GROUP,), jnp.int32),
            pltpu.VMEM((IDX_PER_GROUP, 2), jnp.float32),
            pltpu.VMEM((RW * 2,), jnp.float32),
            pltpu.VMEM((32,), jnp.float32),
            pltpu.SemaphoreType.DMA,
        ],
    )
    def k(t_hbm, x_hbm, b_hbm, out_hbm, idx_v, rows_v, outb_v, bv_v, sem):
        wid = lax.axis_index("s") * NC + lax.axis_index("c")
        pltpu.sync_copy(b_hbm, bv_v)
        bv0 = bv_v[pl.ds(0, 16)]
        bv1 = bv_v[pl.ds(16, 16)]
        iota = lax.iota(jnp.int32, 16)
        even_lane = (iota & 1) == 0
        tail_mask = iota < (SL - 12 * 16)
        col0 = jnp.zeros((16,), jnp.int32)
        col1 = jnp.ones((16,), jnp.int32)
        zero = jnp.zeros((16,), jnp.float32)
        inv_len = 1.0 / SL

        def group_body(g, carry):
            gbase = wid * (RW * SL) + g * IDX_PER_GROUP
            pltpu.sync_copy(x_hbm.at[pl.ds(gbase, IDX_PER_GROUP)], idx_v)
            copies = [
                pltpu.async_copy(
                    t_hbm.at[idx_v.at[pl.ds(jj * HALF, HALF)]],
                    rows_v.at[pl.ds(jj * HALF, HALF)],
                    sem,
                )
                for jj in range(NDMA)
            ]
            for c in copies:
                c.wait()
            res = zero
            for r in range(GROUP):
                rbase = r * SL
                acc0 = zero
                acc1 = zero
                for m in range(12):
                    ridx = iota + (rbase + 16 * m)
                    acc0 = acc0 + plsc.load_gather(rows_v, [ridx, col0])
                    acc1 = acc1 + plsc.load_gather(rows_v, [ridx, col1])
                ridx = jnp.minimum(iota + (rbase + 192), rbase + SL - 1)
                g0 = plsc.load_gather(rows_v, [ridx, col0])
                g1 = plsc.load_gather(rows_v, [ridx, col1])
                acc0 = acc0 + jnp.where(tail_mask, g0, zero)
                acc1 = acc1 + jnp.where(tail_mask, g1, zero)
                l0 = jnp.full((16,), jnp.sum(acc0) * inv_len, jnp.float32) + bv0
                l1 = jnp.full((16,), jnp.sum(acc1) * inv_len, jnp.float32) + bv1
                m_ = jnp.maximum(l0, l1)
                e0 = jnp.exp(l0 - m_)
                e1 = jnp.exp(l1 - m_)
                denom = e0 + e1
                p = jnp.where(even_lane, e0 / denom, e1 / denom)
                res = jnp.where((iota >> 1) == r, p, res)
            outb_v[pl.ds(g * 16, 16)] = res
            return carry

        lax.fori_loop(0, NG, group_body, 0)
        pltpu.sync_copy(outb_v, out_hbm.at[pl.ds(wid * (RW * 2), RW * 2)])

    return k(t, x_flat, bvec)


def kernel(x, emb, W1, b1, W2, b2):
    t = _project(emb, W1, W2)
    beff = W2 @ b1 + b2
    bvec = jnp.concatenate([
        jnp.full((16,), beff[0], jnp.float32),
        jnp.full((16,), beff[1], jnp.float32),
    ])
    x_flat = x.reshape(-1).astype(jnp.int32)
    out_flat = _sc_pool_softmax(t, x_flat, bvec)
    return out_flat.reshape(B, 1, 2)


# speedup vs baseline: 3.3106x; 3.3106x over previous
import functools

import jax
import jax.numpy as jnp
from jax import lax
from jax.experimental import pallas as pl
from jax.experimental.pallas import tpu as pltpu
from jax.experimental.pallas import tpu_sc as plsc

VOCAB = 1000000
D = 64
SL = 200
B = 16384
NC, NS = 2, 16
NW = NC * NS
RW = B // NW
GROUP = 8
NG = RW // GROUP
CHUNK = 80
IDX_PER_GROUP = GROUP * SL
NDMA = IDX_PER_GROUP // CHUNK

_PBLK = 8192
_PGRID = (VOCAB + _PBLK - 1) // _PBLK


def _proj_body(emb_ref, w1_ref, w2_ref, t_ref):
    w = lax.dot_general(w2_ref[...], w1_ref[...], (((1,), (0,)), ((), ())),
                        preferred_element_type=jnp.float32)
    t_ref[...] = lax.dot_general(w, emb_ref[...], (((1,), (1,)), ((), ())),
                                 preferred_element_type=jnp.float32)


def _project(emb, W1, W2):
    return pl.pallas_call(
        _proj_body,
        grid=(_PGRID,),
        in_specs=[
            pl.BlockSpec((_PBLK, D), lambda i: (i, 0)),
            pl.BlockSpec((2 * D, D), lambda i: (0, 0)),
            pl.BlockSpec((2, 2 * D), lambda i: (0, 0)),
        ],
        out_specs=pl.BlockSpec((2, _PBLK), lambda i: (0, i)),
        out_shape=jax.ShapeDtypeStruct((2, VOCAB), jnp.float32),
    )(emb, W1, W2)


def _sc_pool_softmax(t0, t1, x_flat, bvec):
    mesh = plsc.VectorSubcoreMesh(core_axis_name="c", subcore_axis_name="s")

    @functools.partial(
        pl.kernel,
        out_type=jax.ShapeDtypeStruct((B * 2,), jnp.float32),
        mesh=mesh,
        scratch_types=[
            pltpu.VMEM((IDX_PER_GROUP,), jnp.int32),
            pltpu.VMEM((IDX_PER_GROUP + 64,), jnp.float32),
            pltpu.VMEM((IDX_PER_GROUP + 64,), jnp.float32),
            pltpu.VMEM((RW * 2,), jnp.float32),
            pltpu.VMEM((32,), jnp.float32),
            pltpu.SemaphoreType.DMA,
        ],
        compiler_params=pltpu.CompilerParams(needs_layout_passes=False),
    )
    def k(t0_hbm, t1_hbm, x_hbm, b_hbm, out_hbm,
          idx_v, rows0_v, rows1_v, outb_v, bv_v, sem):
        wid = lax.axis_index("s") * NC + lax.axis_index("c")
        pltpu.sync_copy(b_hbm, bv_v)
        bv0 = bv_v[pl.ds(0, 16)]
        bv1 = bv_v[pl.ds(16, 16)]
        iota = lax.iota(jnp.int32, 16)
        even_lane = (iota & 1) == 0
        tail_mask = iota < (SL - 12 * 16)
        zero = jnp.zeros((16,), jnp.float32)
        inv_len = 1.0 / SL

        def group_body(g, carry):
            gbase = wid * (RW * SL) + g * IDX_PER_GROUP
            pltpu.sync_copy(x_hbm.at[pl.ds(gbase, IDX_PER_GROUP)], idx_v)
            copies = []
            for jj in range(NDMA):
                sl_ = pl.ds(jj * CHUNK, CHUNK)
                copies.append(
                    pltpu.async_copy(t0_hbm.at[idx_v.at[sl_]], rows0_v.at[sl_], sem))
                copies.append(
                    pltpu.async_copy(t1_hbm.at[idx_v.at[sl_]], rows1_v.at[sl_], sem))
            for c in copies:
                c.wait()
            res = zero
            for r in range(GROUP):
                rbase = r * SL
                acc0 = zero
                acc1 = zero
                for m in range(12):
                    acc0 = acc0 + rows0_v[pl.ds(rbase + 16 * m, 16)]
                    acc1 = acc1 + rows1_v[pl.ds(rbase + 16 * m, 16)]
                tr0 = rows0_v[pl.ds(rbase + 192, 16)]
                tr1 = rows1_v[pl.ds(rbase + 192, 16)]
                acc0 = acc0 + jnp.where(tail_mask, tr0, zero)
                acc1 = acc1 + jnp.where(tail_mask, tr1, zero)
                l0 = jnp.full((16,), jnp.sum(acc0) * inv_len, jnp.float32) + bv0
                l1 = jnp.full((16,), jnp.sum(acc1) * inv_len, jnp.float32) + bv1
                m_ = jnp.maximum(l0, l1)
                e0 = jnp.exp(l0 - m_)
                e1 = jnp.exp(l1 - m_)
                denom = e0 + e1
                p = jnp.where(even_lane, e0 / denom, e1 / denom)
                res = jnp.where((iota >> 1) == r, p, res)
            outb_v[pl.ds(g * 16, 16)] = res
            return carry

        lax.fori_loop(0, NG, group_body, 0)
        pltpu.sync_copy(outb_v, out_hbm.at[pl.ds(wid * (RW * 2), RW * 2)])

    return k(t0, t1, x_flat, bvec)


def kernel(x, emb, W1, b1, W2, b2):
    tT = _project(emb, W1, W2)
    beff = W2 @ b1 + b2
    bvec = jnp.concatenate([
        jnp.full((16,), beff[0], jnp.float32),
        jnp.full((16,), beff[1], jnp.float32),
    ])
    x_flat = x.reshape(-1).astype(jnp.int32)
    out_flat = _sc_pool_softmax(tT[0], tT[1], x_flat, bvec)
    return out_flat.reshape(B, 1, 2)


# speedup vs baseline: 15.8128x; 4.7765x over previous
import functools

import jax
import jax.numpy as jnp
from jax import lax
from jax.experimental import pallas as pl
from jax.experimental.pallas import tpu as pltpu
from jax.experimental.pallas import tpu_sc as plsc

VOCAB = 1000000
D = 64
SL = 200
B = 16384
NC, NS = 2, 16
NW = NC * NS
RW = B // NW
GROUP = 32
NG = RW // GROUP
CHUNK = 128
IDX_PER_GROUP = GROUP * SL
NDMA = IDX_PER_GROUP // CHUNK

_PBLK = 65536
_PGRID = (VOCAB + _PBLK - 1) // _PBLK


def _proj_body(embt_ref, w1_ref, w2_ref, t_ref):
    w = lax.dot_general(w2_ref[...], w1_ref[...], (((1,), (0,)), ((), ())),
                        preferred_element_type=jnp.float32)
    tt = lax.dot_general(w, embt_ref[...], (((1,), (0,)), ((), ())),
                         preferred_element_type=jnp.float32)
    t0u = lax.convert_element_type(
        lax.bitcast_convert_type(tt[0].astype(jnp.bfloat16), jnp.uint16),
        jnp.uint32)
    t1u = lax.convert_element_type(
        lax.bitcast_convert_type(tt[1].astype(jnp.bfloat16), jnp.uint16),
        jnp.uint32)
    word = jnp.bitwise_or(t0u, jnp.left_shift(t1u, 16))
    t_ref[...] = lax.bitcast_convert_type(word, jnp.int32)


def _project(embT, W1, W2):
    return pl.pallas_call(
        _proj_body,
        grid=(_PGRID,),
        in_specs=[
            pl.BlockSpec((D, _PBLK), lambda i: (0, i)),
            pl.BlockSpec((2 * D, D), lambda i: (0, 0)),
            pl.BlockSpec((2, 2 * D), lambda i: (0, 0)),
        ],
        out_specs=pl.BlockSpec((_PBLK,), lambda i: (i,)),
        out_shape=jax.ShapeDtypeStruct((VOCAB,), jnp.int32),
    )(embT, W1, W2)


def _sc_pool_softmax(t01, x_flat, bvec):
    mesh = plsc.VectorSubcoreMesh(core_axis_name="c", subcore_axis_name="s")

    @functools.partial(
        pl.kernel,
        out_type=jax.ShapeDtypeStruct((B * 2,), jnp.float32),
        mesh=mesh,
        scratch_types=[
            pltpu.VMEM((IDX_PER_GROUP,), jnp.int32),
            pltpu.VMEM((IDX_PER_GROUP,), jnp.int32),
            pltpu.VMEM((IDX_PER_GROUP + 64,), jnp.int32),
            pltpu.VMEM((IDX_PER_GROUP + 64,), jnp.int32),
            pltpu.VMEM((RW * 2,), jnp.float32),
            pltpu.VMEM((32,), jnp.float32),
            pltpu.VMEM_SHARED((VOCAB,), jnp.int32),
            pltpu.VMEM((15616,), jnp.int32),
            pltpu.SemaphoreType.DMA,
            pltpu.SemaphoreType.DMA,
            pltpu.SemaphoreType.DMA,
        ],
        compiler_params=pltpu.CompilerParams(needs_layout_passes=False),
    )
    def k(t_hbm, x_hbm, b_hbm, out_hbm,
          idx0_v, idx1_v, rows0_v, rows1_v, outb_v, bv_v, t_sh, stage_v,
          sem0, sem1, isem):
        wid = lax.axis_index("s") * NC + lax.axis_index("c")
        sid = lax.axis_index("s")
        _SC_ = 15616
        _STG = 4 * _SC_
        sbase = sid * _STG
        for c in range(4):
            cs = pl.ds(sbase + c * _SC_, _SC_)
            pltpu.sync_copy(t_hbm.at[cs], stage_v)
            pltpu.sync_copy(stage_v, t_sh.at[cs])
        rem = VOCAB - 16 * _STG
        def _stage_rem():
            pltpu.sync_copy(t_hbm.at[pl.ds(16 * _STG, rem)],
                            stage_v.at[pl.ds(0, rem)])
            pltpu.sync_copy(stage_v.at[pl.ds(0, rem)],
                            t_sh.at[pl.ds(16 * _STG, rem)])
        pl.when(sid == 0)(_stage_rem)
        pltpu.sync_copy(b_hbm, bv_v)
        plsc.subcore_barrier()
        bv0 = bv_v[pl.ds(0, 16)]
        bv1 = bv_v[pl.ds(16, 16)]
        iota = lax.iota(jnp.int32, 16)
        even_lane = (iota & 1) == 0
        tail_mask = iota < (SL - 12 * 16)
        zero = jnp.zeros((16,), jnp.float32)
        himask = jnp.full((16,), -65536, jnp.int32)
        inv_len = 1.0 / SL
        xbase = wid * (RW * SL)

        def idx_start(g, idxbuf):
            pltpu.async_copy(
                x_hbm.at[pl.ds(xbase + g * IDX_PER_GROUP, IDX_PER_GROUP)],
                idxbuf, isem)

        def idx_wait(idxbuf):
            pltpu.make_async_copy(
                x_hbm.at[pl.ds(xbase, IDX_PER_GROUP)], idxbuf, isem).wait()

        def fire(idxbuf, rowsbuf, sem_):
            for jj in range(NDMA):
                sl_ = pl.ds(jj * CHUNK, CHUNK)
                pltpu.async_copy(t_sh.at[idxbuf.at[sl_]], rowsbuf.at[sl_], sem_)

        def drain(rowsbuf, sem_):
            for jj in range(NDMA):
                pltpu.make_async_copy(
                    t_sh.at[pl.ds(0, CHUNK)],
                    rowsbuf.at[pl.ds(jj * CHUNK, CHUNK)], sem_).wait()

        def accum(rowsbuf, g):
          for h in range(GROUP // 8):
            res = zero
            for r8 in range(8):
                r = h * 8 + r8
                rbase = r * SL
                acc0 = zero
                acc1 = zero
                for m in range(13):
                    w = rowsbuf[pl.ds(rbase + 16 * m, 16)]
                    f0 = plsc.bitcast(jnp.left_shift(w, 16), jnp.float32)
                    f1 = plsc.bitcast(jnp.bitwise_and(w, himask), jnp.float32)
                    if m == 12:
                        f0 = jnp.where(tail_mask, f0, zero)
                        f1 = jnp.where(tail_mask, f1, zero)
                    acc0 = acc0 + f0
                    acc1 = acc1 + f1
                l0 = jnp.full((16,), jnp.sum(acc0) * inv_len, jnp.float32) + bv0
                l1 = jnp.full((16,), jnp.sum(acc1) * inv_len, jnp.float32) + bv1
                m_ = jnp.maximum(l0, l1)
                e0 = jnp.exp(l0 - m_)
                e1 = jnp.exp(l1 - m_)
                denom = e0 + e1
                p = jnp.where(even_lane, e0 / denom, e1 / denom)
                res = jnp.where((iota >> 1) == r8, p, res)
            outb_v[pl.ds(g * (GROUP * 2) + h * 16, 16)] = res

        pltpu.sync_copy(x_hbm.at[pl.ds(xbase, IDX_PER_GROUP)], idx0_v)
        fire(idx0_v, rows0_v, sem0)
        idx_start(1, idx1_v)

        def pair_body(gp, carry):
            a = 2 * gp
            more = gp < (NG // 2 - 1)
            idx_wait(idx1_v)
            fire(idx1_v, rows1_v, sem1)
            drain(rows0_v, sem0)
            pl.when(more)(lambda: idx_start(a + 2, idx0_v))
            accum(rows0_v, a)
            def _fire_next():
                idx_wait(idx0_v)
                fire(idx0_v, rows0_v, sem0)
            pl.when(more)(_fire_next)
            drain(rows1_v, sem1)
            pl.when(more)(lambda: idx_start(a + 3, idx1_v))
            accum(rows1_v, a + 1)
            return carry

        lax.fori_loop(0, NG // 2, pair_body, 0)
        pltpu.sync_copy(outb_v, out_hbm.at[pl.ds(wid * (RW * 2), RW * 2)])

    return k(t01, x_flat, bvec)


def kernel(x, emb, W1, b1, W2, b2):
    t01 = _project(emb.T, W1, W2)
    beff = W2 @ b1 + b2
    bvec = jnp.concatenate([
        jnp.full((16,), beff[0], jnp.float32),
        jnp.full((16,), beff[1], jnp.float32),
    ])
    x_flat = x.reshape(-1).astype(jnp.int32)
    out_flat = _sc_pool_softmax(t01, x_flat, bvec)
    return out_flat.reshape(B, 1, 2)


# speedup vs baseline: 16.4611x; 1.0410x over previous
import functools

import jax
import jax.numpy as jnp
from jax import lax
from jax.experimental import pallas as pl
from jax.experimental.pallas import tpu as pltpu
from jax.experimental.pallas import tpu_sc as plsc

VOCAB = 1000000
D = 64
SL = 200
B = 16384
NC, NS = 2, 16
NW = NC * NS
RW = B // NW
GROUP = 16
NG = RW // GROUP
CHUNK = 128
IDX_PER_GROUP = GROUP * SL
NDMA = IDX_PER_GROUP // CHUNK

_PBLK = 65536
_PGRID = (VOCAB + _PBLK - 1) // _PBLK


def _proj_body(embt_ref, w1_ref, w2_ref, t_ref):
    w = lax.dot_general(w2_ref[...], w1_ref[...], (((1,), (0,)), ((), ())),
                        preferred_element_type=jnp.float32)
    tt = lax.dot_general(w, embt_ref[...], (((1,), (0,)), ((), ())),
                         preferred_element_type=jnp.float32)
    t0u = lax.convert_element_type(
        lax.bitcast_convert_type(tt[0].astype(jnp.bfloat16), jnp.uint16),
        jnp.uint32)
    t1u = lax.convert_element_type(
        lax.bitcast_convert_type(tt[1].astype(jnp.bfloat16), jnp.uint16),
        jnp.uint32)
    word = jnp.bitwise_or(t0u, jnp.left_shift(t1u, 16))
    t_ref[...] = lax.bitcast_convert_type(word, jnp.int32)


def _project(embT, W1, W2):
    return pl.pallas_call(
        _proj_body,
        grid=(_PGRID,),
        in_specs=[
            pl.BlockSpec((D, _PBLK), lambda i: (0, i)),
            pl.BlockSpec((2 * D, D), lambda i: (0, 0)),
            pl.BlockSpec((2, 2 * D), lambda i: (0, 0)),
        ],
        out_specs=pl.BlockSpec((_PBLK,), lambda i: (i,)),
        out_shape=jax.ShapeDtypeStruct((VOCAB,), jnp.int32),
    )(embT, W1, W2)


def _sc_pool_softmax(t01, x_flat, bvec):
    mesh = plsc.VectorSubcoreMesh(core_axis_name="c", subcore_axis_name="s")

    @functools.partial(
        pl.kernel,
        out_type=jax.ShapeDtypeStruct((B * 2,), jnp.float32),
        mesh=mesh,
        scratch_types=[
            pltpu.VMEM((IDX_PER_GROUP,), jnp.int32),
            pltpu.VMEM((IDX_PER_GROUP,), jnp.int32),
            pltpu.VMEM((IDX_PER_GROUP + 64,), jnp.int32),
            pltpu.VMEM((IDX_PER_GROUP + 64,), jnp.int32),
            pltpu.VMEM((RW * 2,), jnp.float32),
            pltpu.VMEM((32,), jnp.float32),
            pltpu.VMEM_SHARED((VOCAB,), jnp.int32),
            pltpu.VMEM((15616,), jnp.int32),
            pltpu.SemaphoreType.DMA,
            pltpu.SemaphoreType.DMA,
            pltpu.SemaphoreType.DMA,
        ],
        compiler_params=pltpu.CompilerParams(needs_layout_passes=False),
    )
    def k(t_hbm, x_hbm, b_hbm, out_hbm,
          idx0_v, idx1_v, rows0_v, rows1_v, outb_v, bv_v, t_sh, stage_v,
          sem0, sem1, isem):
        wid = lax.axis_index("s") * NC + lax.axis_index("c")
        sid = lax.axis_index("s")
        _SC_ = 15616
        _STG = 4 * _SC_
        sbase = sid * _STG
        for c in range(4):
            cs = pl.ds(sbase + c * _SC_, _SC_)
            pltpu.sync_copy(t_hbm.at[cs], stage_v)
            pltpu.sync_copy(stage_v, t_sh.at[cs])
        rem = VOCAB - 16 * _STG
        def _stage_rem():
            pltpu.sync_copy(t_hbm.at[pl.ds(16 * _STG, rem)],
                            stage_v.at[pl.ds(0, rem)])
            pltpu.sync_copy(stage_v.at[pl.ds(0, rem)],
                            t_sh.at[pl.ds(16 * _STG, rem)])
        pl.when(sid == 0)(_stage_rem)
        pltpu.sync_copy(b_hbm, bv_v)
        plsc.subcore_barrier()
        bv0 = bv_v[pl.ds(0, 16)]
        bv1 = bv_v[pl.ds(16, 16)]
        iota = lax.iota(jnp.int32, 16)
        even_lane = (iota & 1) == 0
        tail_mask = iota < (SL - 12 * 16)
        zero = jnp.zeros((16,), jnp.float32)
        himask = jnp.full((16,), -65536, jnp.int32)
        inv_len = 1.0 / SL
        xbase = wid * (RW * SL)

        def idx_start(g, idxbuf):
            pltpu.async_copy(
                x_hbm.at[pl.ds(xbase + g * IDX_PER_GROUP, IDX_PER_GROUP)],
                idxbuf, isem)

        def idx_wait(idxbuf):
            pltpu.make_async_copy(
                x_hbm.at[pl.ds(xbase, IDX_PER_GROUP)], idxbuf, isem).wait()

        def fire(idxbuf, rowsbuf, sem_):
            for jj in range(NDMA):
                sl_ = pl.ds(jj * CHUNK, CHUNK)
                pltpu.async_copy(t_sh.at[idxbuf.at[sl_]], rowsbuf.at[sl_], sem_)

        def drain(rowsbuf, sem_):
            for jj in range(NDMA):
                pltpu.make_async_copy(
                    t_sh.at[pl.ds(0, CHUNK)],
                    rowsbuf.at[pl.ds(jj * CHUNK, CHUNK)], sem_).wait()

        def accum(rowsbuf, g):
          for h in range(GROUP // 8):
            res = zero
            for r8 in range(8):
                r = h * 8 + r8
                rbase = r * SL
                acc0 = zero
                acc1 = zero
                for m in range(13):
                    w = rowsbuf[pl.ds(rbase + 16 * m, 16)]
                    f0 = plsc.bitcast(jnp.left_shift(w, 16), jnp.float32)
                    f1 = plsc.bitcast(jnp.bitwise_and(w, himask), jnp.float32)
                    if m == 12:
                        f0 = jnp.where(tail_mask, f0, zero)
                        f1 = jnp.where(tail_mask, f1, zero)
                    acc0 = acc0 + f0
                    acc1 = acc1 + f1
                l0 = jnp.full((16,), jnp.sum(acc0) * inv_len, jnp.float32) + bv0
                l1 = jnp.full((16,), jnp.sum(acc1) * inv_len, jnp.float32) + bv1
                m_ = jnp.maximum(l0, l1)
                e0 = jnp.exp(l0 - m_)
                e1 = jnp.exp(l1 - m_)
                denom = e0 + e1
                p = jnp.where(even_lane, e0 / denom, e1 / denom)
                res = jnp.where((iota >> 1) == r8, p, res)
            outb_v[pl.ds(g * (GROUP * 2) + h * 16, 16)] = res

        pltpu.sync_copy(x_hbm.at[pl.ds(xbase, IDX_PER_GROUP)], idx0_v)
        fire(idx0_v, rows0_v, sem0)
        idx_start(1, idx1_v)

        def pair_body(gp, carry):
            a = 2 * gp
            more = gp < (NG // 2 - 1)
            idx_wait(idx1_v)
            fire(idx1_v, rows1_v, sem1)
            drain(rows0_v, sem0)
            pl.when(more)(lambda: idx_start(a + 2, idx0_v))
            accum(rows0_v, a)
            def _fire_next():
                idx_wait(idx0_v)
                fire(idx0_v, rows0_v, sem0)
            pl.when(more)(_fire_next)
            drain(rows1_v, sem1)
            pl.when(more)(lambda: idx_start(a + 3, idx1_v))
            accum(rows1_v, a + 1)
            return carry

        lax.fori_loop(0, NG // 2, pair_body, 0)
        pltpu.sync_copy(outb_v, out_hbm.at[pl.ds(wid * (RW * 2), RW * 2)])

    return k(t01, x_flat, bvec)


def kernel(x, emb, W1, b1, W2, b2):
    t01 = _project(emb.T, W1, W2)
    beff = W2 @ b1 + b2
    bvec = jnp.concatenate([
        jnp.full((16,), beff[0], jnp.float32),
        jnp.full((16,), beff[1], jnp.float32),
    ])
    x_flat = x.reshape(-1).astype(jnp.int32)
    out_flat = _sc_pool_softmax(t01, x_flat, bvec)
    return out_flat.reshape(B, 1, 2)
